# Initial kernel scaffold; baseline (speedup 1.0000x reference)
import jax
import jax.numpy as jnp
from jax.experimental import pallas as pl


def kernel(x, edge_index, y, batch, W, b, lin_W, lin_b):
    raise NotImplementedError("write your pallas kernel here")


# speedup vs baseline: 23.6687x; 23.6687x over previous
import functools

import jax
import jax.numpy as jnp
from jax import lax
from jax.experimental import pallas as pl
from jax.experimental.pallas import tpu as pltpu
from jax.experimental.pallas import tpu_sc as plsc

N = 100000
E = 1600000
NUM_FEAT = 11
HIDDEN = 64
NUM_CLASSES = 19
BATCH = 128

FP = 16
BLK = 2048
NB = 49
NPAD = NB * BLK
NTILES = 32
CHUNK = 128
K = 391
EPT = CHUNK * K
ETOT = NTILES * EPT
SLAB = NPAD // 16

_mesh = plsc.VectorSubcoreMesh(core_axis_name="c", subcore_axis_name="s")
_sc_params = pltpu.CompilerParams(use_tc_tiling_on_sc=False)


@functools.partial(
    pl.kernel,
    out_type=jax.ShapeDtypeStruct((2, NPAD), jnp.float32),
    mesh=_mesh,
    scratch_types=[
        pltpu.VMEM((4, CHUNK), jnp.int32),
        pltpu.VMEM((CHUNK,), jnp.float32),
        pltpu.VMEM_SHARED((NPAD,), jnp.float32),
    ],
    compiler_params=_sc_params,
)
def _deg_kernel(dst_hbm, ones_hbm, zeros_hbm, out_hbm, idx_v, ones_v, deg_sp):
    c = lax.axis_index("c")
    s = lax.axis_index("s")
    wid = s * 2 + c
    pltpu.sync_copy(zeros_hbm, deg_sp.at[pl.ds(s * SLAB, SLAB)])
    pltpu.sync_copy(ones_hbm, ones_v)
    plsc.subcore_barrier()
    base = wid * EPT

    def body(j, carry):
        off = base + j * CHUNK
        pltpu.sync_copy(dst_hbm.at[pl.ds(off, CHUNK)], idx_v.at[0])
        pltpu.sync_copy(ones_v, deg_sp.at[idx_v.at[0]], add=True)
        return carry

    lax.fori_loop(0, K, body, 0)
    plsc.subcore_barrier()
    pltpu.sync_copy(
        deg_sp.at[pl.ds(s * SLAB, SLAB)], out_hbm.at[c, pl.ds(s * SLAB, SLAB)]
    )


@functools.partial(
    pl.kernel,
    out_type=jax.ShapeDtypeStruct((2, NPAD, FP), jnp.float32),
    mesh=_mesh,
    scratch_types=[
        pltpu.VMEM((4, CHUNK), jnp.int32),
        pltpu.VMEM((4, CHUNK), jnp.int32),
        pltpu.VMEM((CHUNK, FP), jnp.float32),
        pltpu.VMEM_SHARED((NPAD, FP), jnp.float32),
        pltpu.SemaphoreType.DMA,
    ],
    compiler_params=_sc_params,
)
def _agg_kernel(src_hbm, dst_hbm, xs_hbm, zrows_hbm, out_hbm,
                si_v, di_v, rows_v, agg_sp, sem):
    c = lax.axis_index("c")
    s = lax.axis_index("s")
    wid = s * 2 + c
    pltpu.sync_copy(zrows_hbm, agg_sp.at[pl.ds(s * SLAB, SLAB)])
    plsc.subcore_barrier()
    base = wid * EPT

    def body(j, carry):
        off = base + j * CHUNK
        pltpu.sync_copy(src_hbm.at[pl.ds(off, CHUNK)], si_v.at[0])
        pltpu.sync_copy(dst_hbm.at[pl.ds(off, CHUNK)], di_v.at[0])
        pltpu.async_copy(xs_hbm.at[si_v.at[0]], rows_v, sem).wait()
        pltpu.sync_copy(rows_v, agg_sp.at[di_v.at[0]], add=True)
        return carry

    lax.fori_loop(0, K, body, 0)
    plsc.subcore_barrier()
    pltpu.sync_copy(
        agg_sp.at[pl.ds(s * SLAB, SLAB)], out_hbm.at[c, pl.ds(s * SLAB, SLAB)]
    )


def _prep_body(deg_ref, x_ref, dinv_ref, xs_ref):
    deg = deg_ref[0] + deg_ref[1]
    dinv = jnp.where(deg > 0, lax.rsqrt(jnp.maximum(deg, 1e-12)), 0.0)
    dinv_ref[...] = dinv
    xs_ref[...] = x_ref[...] * dinv


_prep = pl.pallas_call(
    _prep_body,
    grid=(NB,),
    in_specs=[
        pl.BlockSpec((2, BLK, 1), lambda i: (0, i, 0)),
        pl.BlockSpec((BLK, FP), lambda i: (i, 0)),
    ],
    out_specs=[
        pl.BlockSpec((BLK, 1), lambda i: (i, 0)),
        pl.BlockSpec((BLK, FP), lambda i: (i, 0)),
    ],
    out_shape=[
        jax.ShapeDtypeStruct((NPAD, 1), jnp.float32),
        jax.ShapeDtypeStruct((NPAD, FP), jnp.float32),
    ],
)


def _final_body(agg_ref, dinv_ref, batch_ref, w_ref, b_ref, lw_ref, lb_ref,
                out_ref, acc, cnt):
    i = pl.program_id(0)

    @pl.when(i == 0)
    def _():
        acc[...] = jnp.zeros_like(acc)
        cnt[...] = jnp.zeros_like(cnt)

    agg = (agg_ref[0] + agg_ref[1]) * dinv_ref[...]
    h = jnp.maximum(
        jnp.dot(agg, w_ref[...], preferred_element_type=jnp.float32)
        + b_ref[...],
        0.0,
    )
    gids = lax.broadcasted_iota(jnp.int32, (BATCH, BLK), 0)
    mask = (batch_ref[0, 0][None, :] == gids).astype(jnp.float32)
    acc[...] += jnp.dot(mask, h, preferred_element_type=jnp.float32)
    cnt[...] += jnp.sum(mask, axis=1, keepdims=True)

    @pl.when(i == NB - 1)
    def _():
        pooled = acc[...] / jnp.maximum(cnt[...], 1.0)
        out_ref[...] = (
            jnp.dot(pooled, lw_ref[...], preferred_element_type=jnp.float32)
            + lb_ref[...]
        )


_final = pl.pallas_call(
    _final_body,
    grid=(NB,),
    in_specs=[
        pl.BlockSpec((2, BLK, FP), lambda i: (0, i, 0)),
        pl.BlockSpec((BLK, 1), lambda i: (i, 0)),
        pl.BlockSpec((1, 1, BLK), lambda i: (i, 0, 0)),
        pl.BlockSpec((FP, HIDDEN), lambda i: (0, 0)),
        pl.BlockSpec((1, HIDDEN), lambda i: (0, 0)),
        pl.BlockSpec((HIDDEN, NUM_CLASSES), lambda i: (0, 0)),
        pl.BlockSpec((1, NUM_CLASSES), lambda i: (0, 0)),
    ],
    out_specs=pl.BlockSpec((BATCH, NUM_CLASSES), lambda i: (0, 0)),
    out_shape=jax.ShapeDtypeStruct((BATCH, NUM_CLASSES), jnp.float32),
    scratch_shapes=[
        pltpu.VMEM((BATCH, HIDDEN), jnp.float32),
        pltpu.VMEM((BATCH, 1), jnp.float32),
    ],
)


def kernel(x, edge_index, y, batch, W, b, lin_W, lin_b):
    src = edge_index[0]
    dst = edge_index[1]
    pad_e = ETOT - E
    src_p = jnp.concatenate([src, jnp.zeros((pad_e,), jnp.int32)])
    dst_p = jnp.concatenate([dst, jnp.full((pad_e,), N, jnp.int32)])
    x16 = jnp.zeros((NPAD, FP), jnp.float32).at[:N, :NUM_FEAT].set(x)
    batch_p = jnp.concatenate(
        [batch, jnp.full((NPAD - N,), BATCH, jnp.int32)]
    ).reshape(NB, 1, BLK)
    ones_c = jnp.ones((CHUNK,), jnp.float32)
    zer_slab = jnp.zeros((SLAB,), jnp.float32)
    zer_rows = jnp.zeros((SLAB, FP), jnp.float32)

    deg2 = _deg_kernel(dst_p, ones_c, zer_slab)
    dinv, xs = _prep(deg2.reshape(2, NPAD, 1), x16)
    agg2 = _agg_kernel(src_p, dst_p, xs, zer_rows)

    w_pad = jnp.zeros((FP, HIDDEN), jnp.float32).at[:NUM_FEAT].set(W)
    return _final(
        agg2,
        dinv,
        batch_p,
        w_pad,
        b.reshape(1, HIDDEN),
        lin_W,
        lin_b.reshape(1, NUM_CLASSES),
    )
